# Initial kernel scaffold; baseline (speedup 1.0000x reference)
import jax
import jax.numpy as jnp
from jax.experimental import pallas as pl


def kernel(x, edge_index, beta):
    raise NotImplementedError("write your pallas kernel here")


# speedup vs baseline: 6.0694x; 6.0694x over previous
import functools

import jax
import jax.numpy as jnp
from jax import lax
from jax.experimental import pallas as pl
from jax.experimental.pallas import tpu as pltpu
from jax.experimental.pallas import tpu_sc as plsc

N = 10000
E = 320000
D = 128

NC = 2
NS = 16
NW = NC * NS
EPW = E // NW
C = 80
NCHUNK = EPW // C
NPAD = 10240
ROWS_PER_TILE = NPAD // NS
L = 16

_mesh = plsc.VectorSubcoreMesh(core_axis_name="c", subcore_axis_name="s")


def _normalize_tc(x):
  def body(x_ref, xn_ref, n_ref):
    xv = x_ref[...]
    ss = jnp.sum(xv * xv, axis=1, keepdims=True)
    nn = jnp.sqrt(ss)
    inv = 1.0 / jnp.maximum(nn, 1e-30)
    xn_ref[...] = xv * inv
    n_ref[...] = nn
  return pl.pallas_call(
      body,
      out_shape=[
          jax.ShapeDtypeStruct((N, D), jnp.float32),
          jax.ShapeDtypeStruct((N, 1), jnp.float32),
      ],
  )(x)


def _combine_tc(partials):
  def body(p_ref, o_ref):
    o_ref[...] = p_ref[0, :N] + p_ref[1, :N]
  return pl.pallas_call(
      body,
      out_shape=jax.ShapeDtypeStruct((N, D), jnp.float32),
  )(partials)


def _edge_dot(rows_r, rows_c, g, lane_iota):
  res = jnp.zeros((L,), jnp.float32)
  for e in range(L):
    ei = g * L + e
    acc = rows_r[ei, pl.ds(0, L)] * rows_c[ei, pl.ds(0, L)]
    for k in range(1, D // L):
      acc = acc + rows_r[ei, pl.ds(k * L, L)] * rows_c[ei, pl.ds(k * L, L)]
    dsum = jnp.sum(acc)
    res = jnp.where(lane_iota == e, jnp.full((L,), dsum), res)
  return res


def _pass_a(xn, nrm, row, col, betav):
  @functools.partial(
      pl.kernel,
      mesh=_mesh,
      compiler_params=pltpu.CompilerParams(needs_layout_passes=False),
      out_type=[
          jax.ShapeDtypeStruct((E,), jnp.float32),
          jax.ShapeDtypeStruct((N,), jnp.float32),
          jax.ShapeDtypeStruct((N,), jnp.float32),
      ],
      scratch_types=[
          pltpu.VMEM((C,), jnp.int32),
          pltpu.VMEM((C,), jnp.int32),
          pltpu.VMEM((C, D), jnp.float32),
          pltpu.VMEM((C, D), jnp.float32),
          pltpu.VMEM((C,), jnp.float32),
          pltpu.VMEM((N,), jnp.float32),
          pltpu.VMEM((L,), jnp.float32),
          pltpu.VMEM((N,), jnp.float32),
          pltpu.VMEM_SHARED((N,), jnp.float32),
          pltpu.SemaphoreType.DMA,
          pltpu.SemaphoreType.DMA,
      ],
  )
  def k2(xn_hbm, nrm_hbm, row_hbm, col_hbm, beta_hbm,
         w_hbm, denom0_hbm, denom1_hbm,
         idx_r_v, idx_c_v, rows_r_v, rows_c_v, wbuf_v, nrm_v, beta_v,
         zeros_v, denom_sh, sem_r, sem_c):
    cid = lax.axis_index("c")
    sid = lax.axis_index("s")
    wid = cid * NS + sid

    pltpu.sync_copy(nrm_hbm, nrm_v)
    pltpu.sync_copy(beta_hbm, beta_v)

    def zbody(i, _):
      zeros_v[pl.ds(i * L, L)] = jnp.zeros((L,), jnp.float32)
      return 0
    lax.fori_loop(0, N // L, zbody, 0)

    @pl.when(sid == 0)
    def _():
      pltpu.sync_copy(zeros_v, denom_sh)
    plsc.subcore_barrier()

    bvec = beta_v[...]

    def chunk_body(i, _):
      base = wid * EPW + i * C
      pltpu.sync_copy(row_hbm.at[pl.ds(base, C)], idx_r_v)
      pltpu.sync_copy(col_hbm.at[pl.ds(base, C)], idx_c_v)
      cp_r = pltpu.async_copy(xn_hbm.at[idx_r_v], rows_r_v, sem_r)
      cp_c = pltpu.async_copy(xn_hbm.at[idx_c_v], rows_c_v, sem_c)
      cp_r.wait()
      cp_c.wait()
      lane_iota = lax.iota(jnp.int32, L)
      for g in range(C // L):
        inner = _edge_dot(rows_r_v, rows_c_v, g, lane_iota)
        ir = idx_r_v[pl.ds(g * L, L)]
        ic = idx_c_v[pl.ds(g * L, L)]
        n_r = plsc.load_gather(nrm_v, [ir])
        n_c = plsc.load_gather(nrm_v, [ic])
        nprod = n_r * n_c
        f = nprod / (nprod + 1e-7)
        w = jnp.exp(inner * f * bvec)
        wbuf_v[pl.ds(g * L, L)] = w
      pltpu.sync_copy(wbuf_v, w_hbm.at[pl.ds(base, C)])
      pltpu.sync_copy(wbuf_v, denom_sh.at[idx_r_v], add=True)
      return 0

    lax.fori_loop(0, NCHUNK, chunk_body, 0)

    plsc.subcore_barrier()
    @pl.when(jnp.logical_and(sid == 0, cid == 0))
    def _():
      pltpu.sync_copy(denom_sh, denom0_hbm)
    @pl.when(jnp.logical_and(sid == 0, cid == 1))
    def _():
      pltpu.sync_copy(denom_sh, denom1_hbm)

  return k2(xn, nrm, row, col, betav)


def _pass_b(xn, nrm, row, col, w, denom0, denom1, znd):
  @functools.partial(
      pl.kernel,
      mesh=_mesh,
      compiler_params=pltpu.CompilerParams(needs_layout_passes=False),
      out_type=jax.ShapeDtypeStruct((NC, NPAD, D), jnp.float32),
      scratch_types=[
          pltpu.VMEM((C,), jnp.int32),
          pltpu.VMEM((C,), jnp.int32),
          pltpu.VMEM((C, D), jnp.float32),
          pltpu.VMEM((C,), jnp.float32),
          pltpu.VMEM((N,), jnp.float32),
          pltpu.VMEM((N,), jnp.float32),
          pltpu.VMEM_SHARED((NPAD, D), jnp.float32),
          pltpu.SemaphoreType.DMA,
      ],
  )
  def k3(xn_hbm, nrm_hbm, row_hbm, col_hbm, w_hbm, denom0_hbm, denom1_hbm,
         znd_hbm, out_hbm,
         idx_r_v, idx_c_v, rows_v, wv_v, h_v, tmp_v, out_sh, sem):
    cid = lax.axis_index("c")
    sid = lax.axis_index("s")
    wid = cid * NS + sid

    pltpu.sync_copy(denom0_hbm, h_v)
    pltpu.sync_copy(denom1_hbm, tmp_v)
    def hbody1(i, _):
      sl = pl.ds(i * L, L)
      h_v[sl] = h_v[sl] + tmp_v[sl] + 1e-16
      return 0
    lax.fori_loop(0, N // L, hbody1, 0)
    pltpu.sync_copy(nrm_hbm, tmp_v)
    def hbody2(i, _):
      sl = pl.ds(i * L, L)
      h_v[sl] = tmp_v[sl] / h_v[sl]
      return 0
    lax.fori_loop(0, N // L, hbody2, 0)

    sl_rows = pl.ds(sid * ROWS_PER_TILE, ROWS_PER_TILE)
    pltpu.sync_copy(znd_hbm.at[sl_rows], out_sh.at[sl_rows])
    plsc.subcore_barrier()

    def chunk_body(i, _):
      base = wid * EPW + i * C
      pltpu.sync_copy(row_hbm.at[pl.ds(base, C)], idx_r_v)
      pltpu.sync_copy(col_hbm.at[pl.ds(base, C)], idx_c_v)
      pltpu.sync_copy(w_hbm.at[pl.ds(base, C)], wv_v)
      pltpu.async_copy(xn_hbm.at[idx_r_v], rows_v, sem).wait()
      for g in range(C // L):
        ir = idx_r_v[pl.ds(g * L, L)]
        s_vec = wv_v[pl.ds(g * L, L)] * plsc.load_gather(h_v, [ir])
        for e in range(L):
          ei = g * L + e
          sp = jnp.full((L,), s_vec[e])
          for k in range(D // L):
            sl = pl.ds(k * L, L)
            rows_v[ei, sl] = rows_v[ei, sl] * sp
      pltpu.sync_copy(rows_v, out_sh.at[idx_c_v], add=True)
      return 0

    lax.fori_loop(0, NCHUNK, chunk_body, 0)

    plsc.subcore_barrier()
    pltpu.sync_copy(
        out_sh.at[pl.ds(sid * ROWS_PER_TILE, ROWS_PER_TILE)],
        out_hbm.at[cid, pl.ds(sid * ROWS_PER_TILE, ROWS_PER_TILE)])

  return k3(xn, nrm, row, col, w, denom0, denom1, znd)


def kernel(x, edge_index, beta):
  row = edge_index[0]
  col = edge_index[1]
  xn, n2 = _normalize_tc(x)
  nrm = n2.reshape(N)
  betav = jnp.full((L,), beta, jnp.float32)
  w, denom0, denom1 = _pass_a(xn, nrm, row, col, betav)
  znd = jnp.zeros((NPAD, D), jnp.float32)
  partials = _pass_b(xn, nrm, row, col, w, denom0, denom1, znd)
  return _combine_tc(partials)
